# Initial kernel scaffold; baseline (speedup 1.0000x reference)
import jax
import jax.numpy as jnp
from jax.experimental import pallas as pl


def kernel(user_feat, item_embed, edge_follows, edge_clicks, edge_clicked_by, W1_follows, b1_follows, W2_follows, b2_follows, W1_clicks, b1_clicks, W2_clicks, b2_clicks, W1_clicked_by, b1_clicked_by, W2_clicked_by, b2_clicked_by, W_out, b_out):
    raise NotImplementedError("write your pallas kernel here")


# speedup vs baseline: 5.0690x; 5.0690x over previous
import jax
import jax.numpy as jnp
from jax import lax
from jax.experimental import pallas as pl
from jax.experimental.pallas import tpu as pltpu
from jax.experimental.pallas import tpu_sc as plsc

N = 10000
E = 320000
IN = 128
HID = 128
OUT = 64
W1PAD = 144

NC = 2
NS = 16
NW = NC * NS
EW = E // NW
CH = 80
NCHUNK = EW // CH
RPT = N // NS
ZR = 125

R = 1000
GB = N // R


def _make_sc_agg(num_tables, etype_table, width):
    ne = len(etype_table)
    lanes = width // 16
    mesh = plsc.VectorSubcoreMesh(core_axis_name="c", subcore_axis_name="s")

    def body(*refs):
        tables = refs[:num_tables]
        srcs = refs[num_tables:num_tables + ne]
        dsts = refs[num_tables + ne:num_tables + 2 * ne]
        out, acc, sidx, didx, rows = refs[num_tables + 2 * ne:]
        c = lax.axis_index("c")
        s = lax.axis_index("s")
        w = c * NS + s
        nz = N // CH

        for e in range(ne):
            def _zero(i, carry):
                for j in range(lanes):
                    rows[i, pl.ds(j * 16, 16)] = jnp.zeros((16,), jnp.float32)
                return carry
            lax.fori_loop(0, CH, _zero, 0)

            def _zacc(k, carry):
                blk = k * NS + s
                @pl.when(blk < nz)
                def _():
                    pltpu.sync_copy(rows, acc.at[pl.ds(blk * CH, CH)])
                return carry
            lax.fori_loop(0, (nz + NS - 1) // NS, _zacc, 0)
            plsc.subcore_barrier()

            table = tables[etype_table[e]]
            pltpu.sync_copy(srcs[e].at[w], sidx)
            pltpu.sync_copy(dsts[e].at[w], didx)

            def _step(i, carry):
                pltpu.sync_copy(table.at[sidx.at[i]], rows)
                pltpu.sync_copy(rows, acc.at[didx.at[i]], add=True)
                return carry
            lax.fori_loop(0, NCHUNK, _step, 0)
            plsc.subcore_barrier()

            def _wout(k, carry):
                blk = k * NS + s
                @pl.when(blk < nz)
                def _():
                    pltpu.sync_copy(acc.at[pl.ds(blk * CH, CH)],
                                    out.at[e, c, pl.ds(blk * CH, CH)])
                return carry
            lax.fori_loop(0, (nz + NS - 1) // NS, _wout, 0)
            plsc.subcore_barrier()

    return pl.kernel(
        body,
        out_type=jax.ShapeDtypeStruct((ne, NC, N, width), jnp.float32),
        mesh=mesh,
        compiler_params=pltpu.CompilerParams(use_tc_tiling_on_sc=False),
        scratch_types=[
            pltpu.VMEM_SHARED((N, width), jnp.float32),
            pltpu.VMEM((NCHUNK, CH), jnp.int32),
            pltpu.VMEM((NCHUNK, CH), jnp.int32),
            pltpu.VMEM((CH, width), jnp.float32),
        ],
    )


def _mid_body(sf0, sf1, scb0, scb1, sc0, sc1,
              w1f, b1f, w1cb, b1cb, w1c, b1c, w2f, w2cb, wo, b2f, b2cb, bo,
              yu, yi, aux):
    sf = sf0[...] + sf1[...]
    scb = scb0[...] + scb1[...]
    sc = sc0[...] + sc1[...]
    cf = sf[:, IN:IN + 1]
    ccb = scb[:, IN:IN + 1]
    cc = sc[:, IN:IN + 1]
    invf = 1.0 / jnp.maximum(cf, 1.0)
    invcb = 1.0 / jnp.maximum(ccb, 1.0)
    invc = 1.0 / jnp.maximum(cc, 1.0)
    hasf = (cf > 0).astype(jnp.float32)
    hascb = (ccb > 0).astype(jnp.float32)
    hasc = (cc > 0).astype(jnp.float32)
    mf = sf[:, :IN] * invf
    mcb = scb[:, :IN] * invcb
    mc = sc[:, :IN] * invc
    hu = (jnp.dot(mf, w1f[...], preferred_element_type=jnp.float32)
          + jnp.dot(mcb, w1cb[...], preferred_element_type=jnp.float32)
          + hasf * b1f[...][None, :] + hascb * b1cb[...][None, :])
    hu = jnp.where(hu >= 0, hu, 0.01 * hu)
    hi = (jnp.dot(mc, w1c[...], preferred_element_type=jnp.float32)
          + hasc * b1c[...][None, :])
    hi = jnp.where(hi >= 0, hi, 0.01 * hi)
    yu[...] = jnp.dot(jnp.dot(hu, w2f[...], preferred_element_type=jnp.float32),
                      wo[...], preferred_element_type=jnp.float32)
    yi[...] = jnp.dot(jnp.dot(hi, w2cb[...], preferred_element_type=jnp.float32),
                      wo[...], preferred_element_type=jnp.float32)
    bias = (bo[...][None, :]
            + hasf * jnp.dot(b2f[...][None, :], wo[...],
                             preferred_element_type=jnp.float32)
            + hascb * jnp.dot(b2cb[...][None, :], wo[...],
                              preferred_element_type=jnp.float32))
    aux[...] = jnp.concatenate(
        [bias, invf, invcb, jnp.zeros((R, IN - OUT - 2), jnp.float32)], axis=1)


def _out_body(sf20, sf21, scb20, scb21, auxr, outr):
    s = sf20[...] + sf21[...]
    t = scb20[...] + scb21[...]
    outr[...] = (s * auxr[:, OUT:OUT + 1] + t * auxr[:, OUT + 1:OUT + 2]
                 + auxr[:, :OUT])


def _row_spec(width):
    return pl.BlockSpec((R, width), lambda i: (i, 0))


def _full2d(a, b):
    return pl.BlockSpec((a, b), lambda i: (0, 0))


def _full1d(a):
    return pl.BlockSpec((a,), lambda i: (0,))


def _mid_call(*args):
    return pl.pallas_call(
        _mid_body,
        grid=(GB,),
        in_specs=[_row_spec(W1PAD)] * 6
        + [_full2d(IN, HID), _full1d(HID), _full2d(IN, HID), _full1d(HID),
           _full2d(IN, HID), _full1d(HID), _full2d(HID, HID),
           _full2d(HID, HID), _full2d(HID, OUT), _full1d(HID), _full1d(HID),
           _full1d(OUT)],
        out_specs=[_row_spec(OUT), _row_spec(OUT), _row_spec(IN)],
        out_shape=[jax.ShapeDtypeStruct((N, OUT), jnp.float32),
                   jax.ShapeDtypeStruct((N, OUT), jnp.float32),
                   jax.ShapeDtypeStruct((N, IN), jnp.float32)],
    )(*args)


def _out_call(*args):
    return pl.pallas_call(
        _out_body,
        grid=(GB,),
        in_specs=[_row_spec(OUT)] * 4 + [_row_spec(IN)],
        out_specs=_row_spec(OUT),
        out_shape=jax.ShapeDtypeStruct((N, OUT), jnp.float32),
    )(*args)


def _prep_edges(edge):
    src = edge[0].astype(jnp.int32).reshape(NW, NCHUNK, CH)
    dst = edge[1].astype(jnp.int32).reshape(NW, NCHUNK, CH)
    return src, dst


def kernel(user_feat, item_embed, edge_follows, edge_clicks, edge_clicked_by,
           W1_follows, b1_follows, W2_follows, b2_follows,
           W1_clicks, b1_clicks, W2_clicks, b2_clicks,
           W1_clicked_by, b1_clicked_by, W2_clicked_by, b2_clicked_by,
           W_out, b_out):
    ones = jnp.ones((N, 1), jnp.float32)
    zpad = jnp.zeros((N, W1PAD - IN - 1), jnp.float32)
    u_tab = jnp.concatenate([user_feat, ones, zpad], axis=1)
    i_tab = jnp.concatenate([item_embed, ones, zpad], axis=1)

    efs, efd = _prep_edges(edge_follows)
    ecs, ecd = _prep_edges(edge_clicks)
    ecbs, ecbd = _prep_edges(edge_clicked_by)

    s1 = _make_sc_agg(2, (0, 0, 1), W1PAD)(
        u_tab, i_tab, efs, ecs, ecbs, efd, ecd, ecbd)
    yu, yi, aux = _mid_call(
        s1[0, 0], s1[0, 1], s1[2, 0], s1[2, 1], s1[1, 0], s1[1, 1],
        W1_follows, b1_follows, W1_clicked_by, b1_clicked_by,
        W1_clicks, b1_clicks, W2_follows, W2_clicked_by, W_out,
        b2_follows, b2_clicked_by, b_out)
    s2 = _make_sc_agg(2, (0, 1), OUT)(yu, yi, efs, ecbs, efd, ecbd)
    return _out_call(s2[0, 0], s2[0, 1], s2[1, 0], s2[1, 1], aux)
